# baseline (speedup 1.0000x reference)
import jax
import jax.numpy as jnp
from jax import lax
from jax.experimental import pallas as pl
from jax.experimental.pallas import tpu as pltpu
from jax.experimental.pallas import tpu_sc as plsc

_NC, _NS, _L = 2, 16, 16
_NW = _NC * _NS
_B, _P, _D = 64, 577, 768
_NBG = 4
_NRB = 8
_BPW = _B // _NBG
_RPB = (_P - 1) // _NRB
_NBUF = 3
_RPS = _RPB // _NBUF
_NCOL = _D // _L


def _sc_body(patch_hbm, table_hbm, out_hbm, tab_v, slab_v, ttab_v, tail_v):
    w = lax.axis_index("s") * _NC + lax.axis_index("c")
    bg = w // _NRB
    rb = w % _NRB
    b0 = bg * _BPW
    p0 = rb * _RPB

    pltpu.sync_copy(table_hbm.at[pl.ds(p0, _RPB)], tab_v)

    def batch_body(i, carry):
        b = b0 + i
        for k in range(_NBUF):
            pltpu.sync_copy(
                patch_hbm.at[pl.ds(b, 1), pl.ds(p0 + k * _RPS, _RPS)],
                slab_v,
            )

            def row_body(r, c2):
                for c in range(_NCOL):
                    sl = pl.ds(c * _L, _L)
                    slab_v[0, r, sl] = slab_v[0, r, sl] + tab_v[k * _RPS + r, sl]
                return c2

            lax.fori_loop(0, _RPS, row_body, 0)
            pltpu.sync_copy(
                slab_v,
                out_hbm.at[pl.ds(b, 1), pl.ds(p0 + k * _RPS, _RPS)],
            )
        return carry

    lax.fori_loop(0, _BPW, batch_body, 0)

    pltpu.sync_copy(table_hbm.at[pl.ds(_P - 1, 1)], ttab_v)
    for k in range(2):
        b = w * 2 + k
        pltpu.sync_copy(
            patch_hbm.at[pl.ds(b, 1), pl.ds(_P - 1, 1)], tail_v.at[k]
        )
        for c in range(_NCOL):
            sl = pl.ds(c * _L, _L)
            tail_v[k, 0, 0, sl] = tail_v[k, 0, 0, sl] + ttab_v[0, sl]
        pltpu.sync_copy(
            tail_v.at[k], out_hbm.at[pl.ds(b, 1), pl.ds(_P - 1, 1)]
        )


@jax.jit
def _sc_call(patch, pos_table):
    mesh = plsc.VectorSubcoreMesh(core_axis_name="c", subcore_axis_name="s")
    f = pl.kernel(
        _sc_body,
        out_type=jax.ShapeDtypeStruct((_B, _P, _D), jnp.float32),
        mesh=mesh,
        scratch_types=[
            pltpu.VMEM((_RPB, _D), jnp.float32),
            pltpu.VMEM((1, _RPS, _D), jnp.float32),
            pltpu.VMEM((1, _D), jnp.float32),
            pltpu.VMEM((2, 1, 1, _D), jnp.float32),
        ],
    )
    return f(patch, pos_table)


def kernel(patch, pos_table):
    return _sc_call(patch, pos_table)


# speedup vs baseline: 1.1385x; 1.1385x over previous
import jax
import jax.numpy as jnp
from jax import lax
from jax.experimental import pallas as pl
from jax.experimental.pallas import tpu as pltpu
from jax.experimental.pallas import tpu_sc as plsc

_NC, _NS, _L = 2, 16, 16
_NW = _NC * _NS
_B, _P, _D = 64, 577, 768
_NBG = 4
_NRB = 8
_BPW = _B // _NBG
_RPB = (_P - 1) // _NRB
_NBUF = 3
_RPS = _RPB // _NBUF
_NCOL = _D // _L


def _in_copy(patch_hbm, ring_v, sems, b, p0, k):
    return pltpu.make_async_copy(
        patch_hbm.at[pl.ds(b, 1), pl.ds(p0 + k * _RPS, _RPS)],
        ring_v.at[k], sems[k],
    )


def _out_copy(out_hbm, ring_v, sems, b, p0, k):
    return pltpu.make_async_copy(
        ring_v.at[k],
        out_hbm.at[pl.ds(b, 1), pl.ds(p0 + k * _RPS, _RPS)],
        sems[k],
    )


def _add_slab(ring_v, tab_v, k):
    def row_body(r, carry):
        for c in range(_NCOL):
            sl = pl.ds(c * _L, _L)
            ring_v[k, 0, r, sl] = ring_v[k, 0, r, sl] + tab_v[k * _RPS + r, sl]
        return carry

    lax.fori_loop(0, _RPS, row_body, 0)


def _sc_body(patch_hbm, table_hbm, out_hbm, tab_v, ring_v, ttab_v, tail_v,
             in_sems, out_sems):
    w = lax.axis_index("s") * _NC + lax.axis_index("c")
    bg = w // _NRB
    rb = w % _NRB
    b0 = bg * _BPW
    p0 = rb * _RPB

    pltpu.sync_copy(table_hbm.at[pl.ds(p0, _RPB)], tab_v)

    for k in range(_NBUF):
        _in_copy(patch_hbm, ring_v, in_sems, b0, p0, k).start()

    def batch_body(i, carry):
        b = b0 + i
        for k in range(_NBUF):
            _in_copy(patch_hbm, ring_v, in_sems, b, p0, k).wait()
            _add_slab(ring_v, tab_v, k)
            _out_copy(out_hbm, ring_v, out_sems, b, p0, k).start()
        for k in range(_NBUF):
            _out_copy(out_hbm, ring_v, out_sems, b, p0, k).wait()
            _in_copy(patch_hbm, ring_v, in_sems, b + 1, p0, k).start()
        return carry

    lax.fori_loop(0, _BPW - 1, batch_body, 0)

    bl = b0 + _BPW - 1
    for k in range(_NBUF):
        _in_copy(patch_hbm, ring_v, in_sems, bl, p0, k).wait()
        _add_slab(ring_v, tab_v, k)
        _out_copy(out_hbm, ring_v, out_sems, bl, p0, k).start()
    for k in range(_NBUF):
        _out_copy(out_hbm, ring_v, out_sems, bl, p0, k).wait()

    pltpu.sync_copy(table_hbm.at[pl.ds(_P - 1, 1)], ttab_v)
    for k in range(2):
        b = w * 2 + k
        pltpu.sync_copy(
            patch_hbm.at[pl.ds(b, 1), pl.ds(_P - 1, 1)], tail_v.at[k]
        )
        for c in range(_NCOL):
            sl = pl.ds(c * _L, _L)
            tail_v[k, 0, 0, sl] = tail_v[k, 0, 0, sl] + ttab_v[0, sl]
        pltpu.sync_copy(
            tail_v.at[k], out_hbm.at[pl.ds(b, 1), pl.ds(_P - 1, 1)]
        )


@jax.jit
def _sc_call(patch, pos_table):
    mesh = plsc.VectorSubcoreMesh(core_axis_name="c", subcore_axis_name="s")
    f = pl.kernel(
        lambda *refs: _sc_body(
            refs[0], refs[1], refs[2], refs[3], refs[4], refs[5], refs[6],
            list(refs[7:7 + _NBUF]), list(refs[7 + _NBUF:7 + 2 * _NBUF]),
        ),
        out_type=jax.ShapeDtypeStruct((_B, _P, _D), jnp.float32),
        mesh=mesh,
        scratch_types=(
            [
                pltpu.VMEM((_RPB, _D), jnp.float32),
                pltpu.VMEM((_NBUF, 1, _RPS, _D), jnp.float32),
                pltpu.VMEM((1, _D), jnp.float32),
                pltpu.VMEM((2, 1, 1, _D), jnp.float32),
            ]
            + [pltpu.SemaphoreType.DMA] * (2 * _NBUF)
        ),
    )
    return f(patch, pos_table)


def kernel(patch, pos_table):
    return _sc_call(patch, pos_table)


# speedup vs baseline: 1.5335x; 1.3470x over previous
import jax
import jax.numpy as jnp
from jax import lax
from jax.experimental import pallas as pl
from jax.experimental.pallas import tpu as pltpu
from jax.experimental.pallas import tpu_sc as plsc

_NC, _NS, _L = 2, 16, 16
_NW = _NC * _NS
_B, _P, _D = 64, 577, 768
_NBG = 4
_NRB = 8
_BPW = _B // _NBG
_RPB = (_P - 1) // _NRB
_NBUF = 3
_RPS = _RPB // _NBUF
_NCOL = _D // _L


def _in_copy(patch_hbm, ring_v, sems, b, p0, k):
    return pltpu.make_async_copy(
        patch_hbm.at[pl.ds(b, 1), pl.ds(p0 + k * _RPS, _RPS)],
        ring_v.at[k], sems[k],
    )


def _out_copy(out_hbm, ring_v, sems, b, p0, k):
    return pltpu.make_async_copy(
        ring_v.at[k],
        out_hbm.at[pl.ds(b, 1), pl.ds(p0 + k * _RPS, _RPS)],
        sems[k],
    )


def _add_slab(ring_v, tab_v, k):
    @plsc.parallel_loop(0, _RPS, 1, unroll=2)
    def row_body(r):
        for c in range(_NCOL):
            sl = pl.ds(c * _L, _L)
            ring_v[k, 0, r, sl] = ring_v[k, 0, r, sl] + tab_v[k * _RPS + r, sl]


def _sc_body(patch_hbm, table_hbm, out_hbm, tab_v, ring_v, ttab_v, tail_v,
             in_sems, out_sems):
    w = lax.axis_index("s") * _NC + lax.axis_index("c")
    bg = w // _NRB
    rb = w % _NRB
    b0 = bg * _BPW
    p0 = rb * _RPB

    pltpu.sync_copy(table_hbm.at[pl.ds(p0, _RPB)], tab_v)

    for k in range(_NBUF):
        _in_copy(patch_hbm, ring_v, in_sems, b0, p0, k).start()

    def batch_body(i, carry):
        b = b0 + i
        for k in range(_NBUF):
            _in_copy(patch_hbm, ring_v, in_sems, b, p0, k).wait()
            _add_slab(ring_v, tab_v, k)
            _out_copy(out_hbm, ring_v, out_sems, b, p0, k).start()
        for k in range(_NBUF):
            _out_copy(out_hbm, ring_v, out_sems, b, p0, k).wait()
            _in_copy(patch_hbm, ring_v, in_sems, b + 1, p0, k).start()
        return carry

    lax.fori_loop(0, _BPW - 1, batch_body, 0)

    bl = b0 + _BPW - 1
    for k in range(_NBUF):
        _in_copy(patch_hbm, ring_v, in_sems, bl, p0, k).wait()
        _add_slab(ring_v, tab_v, k)
        _out_copy(out_hbm, ring_v, out_sems, bl, p0, k).start()
    for k in range(_NBUF):
        _out_copy(out_hbm, ring_v, out_sems, bl, p0, k).wait()

    pltpu.sync_copy(table_hbm.at[pl.ds(_P - 1, 1)], ttab_v)
    for k in range(2):
        b = w * 2 + k
        pltpu.sync_copy(
            patch_hbm.at[pl.ds(b, 1), pl.ds(_P - 1, 1)], tail_v.at[k]
        )
        for c in range(_NCOL):
            sl = pl.ds(c * _L, _L)
            tail_v[k, 0, 0, sl] = tail_v[k, 0, 0, sl] + ttab_v[0, sl]
        pltpu.sync_copy(
            tail_v.at[k], out_hbm.at[pl.ds(b, 1), pl.ds(_P - 1, 1)]
        )


@jax.jit
def _sc_call(patch, pos_table):
    mesh = plsc.VectorSubcoreMesh(core_axis_name="c", subcore_axis_name="s")
    f = pl.kernel(
        lambda *refs: _sc_body(
            refs[0], refs[1], refs[2], refs[3], refs[4], refs[5], refs[6],
            list(refs[7:7 + _NBUF]), list(refs[7 + _NBUF:7 + 2 * _NBUF]),
        ),
        out_type=jax.ShapeDtypeStruct((_B, _P, _D), jnp.float32),
        mesh=mesh,
        scratch_types=(
            [
                pltpu.VMEM((_RPB, _D), jnp.float32),
                pltpu.VMEM((_NBUF, 1, _RPS, _D), jnp.float32),
                pltpu.VMEM((1, _D), jnp.float32),
                pltpu.VMEM((2, 1, 1, _D), jnp.float32),
            ]
            + [pltpu.SemaphoreType.DMA] * (2 * _NBUF)
        ),
    )
    return f(patch, pos_table)


def kernel(patch, pos_table):
    return _sc_call(patch, pos_table)


# speedup vs baseline: 1.5893x; 1.0363x over previous
import jax
import jax.numpy as jnp
from jax import lax
from jax.experimental import pallas as pl
from jax.experimental.pallas import tpu as pltpu
from jax.experimental.pallas import tpu_sc as plsc

_NC, _NS, _L = 2, 16, 16
_NW = _NC * _NS
_B, _P, _D = 64, 577, 768
_NBG = 4
_NRB = 8
_BPW = _B // _NBG
_RPB = (_P - 1) // _NRB
_NBUF = 3
_RPS = _RPB // _NBUF
_NCOL = _D // _L


def _in_copy(patch_hbm, ring_v, sems, b, p0, k):
    return pltpu.make_async_copy(
        patch_hbm.at[pl.ds(b, 1), pl.ds(p0 + k * _RPS, _RPS)],
        ring_v.at[k], sems[k],
    )


def _out_copy(out_hbm, ring_v, sems, b, p0, k):
    return pltpu.make_async_copy(
        ring_v.at[k],
        out_hbm.at[pl.ds(b, 1), pl.ds(p0 + k * _RPS, _RPS)],
        sems[k],
    )


def _add_slab(ring_v, tab_v, k):
    @plsc.parallel_loop(0, _RPS, 1, unroll=2)
    def row_body(r):
        for c0 in range(0, _NCOL, 8):
            vals = []
            for c in range(c0, c0 + 8):
                sl = pl.ds(c * _L, _L)
                vals.append(ring_v[k, 0, r, sl] + tab_v[k * _RPS + r, sl])
            for j, c in enumerate(range(c0, c0 + 8)):
                sl = pl.ds(c * _L, _L)
                ring_v[k, 0, r, sl] = vals[j]


def _sc_body(patch_hbm, table_hbm, out_hbm, tab_v, ring_v, ttab_v, tail_v,
             in_sems, out_sems):
    w = lax.axis_index("s") * _NC + lax.axis_index("c")
    bg = w // _NRB
    rb = w % _NRB
    b0 = bg * _BPW
    p0 = rb * _RPB

    pltpu.sync_copy(table_hbm.at[pl.ds(p0, _RPB)], tab_v)

    for k in range(_NBUF):
        _in_copy(patch_hbm, ring_v, in_sems, b0, p0, k).start()

    def batch_body(i, carry):
        b = b0 + i
        for k in range(_NBUF):
            _in_copy(patch_hbm, ring_v, in_sems, b, p0, k).wait()
            _add_slab(ring_v, tab_v, k)
            _out_copy(out_hbm, ring_v, out_sems, b, p0, k).start()
        for k in range(_NBUF):
            _out_copy(out_hbm, ring_v, out_sems, b, p0, k).wait()
            _in_copy(patch_hbm, ring_v, in_sems, b + 1, p0, k).start()
        return carry

    lax.fori_loop(0, _BPW - 1, batch_body, 0)

    bl = b0 + _BPW - 1
    for k in range(_NBUF):
        _in_copy(patch_hbm, ring_v, in_sems, bl, p0, k).wait()
        _add_slab(ring_v, tab_v, k)
        _out_copy(out_hbm, ring_v, out_sems, bl, p0, k).start()
    for k in range(_NBUF):
        _out_copy(out_hbm, ring_v, out_sems, bl, p0, k).wait()

    pltpu.sync_copy(table_hbm.at[pl.ds(_P - 1, 1)], ttab_v)
    for k in range(2):
        b = w * 2 + k
        pltpu.sync_copy(
            patch_hbm.at[pl.ds(b, 1), pl.ds(_P - 1, 1)], tail_v.at[k]
        )
        for c in range(_NCOL):
            sl = pl.ds(c * _L, _L)
            tail_v[k, 0, 0, sl] = tail_v[k, 0, 0, sl] + ttab_v[0, sl]
        pltpu.sync_copy(
            tail_v.at[k], out_hbm.at[pl.ds(b, 1), pl.ds(_P - 1, 1)]
        )


@jax.jit
def _sc_call(patch, pos_table):
    mesh = plsc.VectorSubcoreMesh(core_axis_name="c", subcore_axis_name="s")
    f = pl.kernel(
        lambda *refs: _sc_body(
            refs[0], refs[1], refs[2], refs[3], refs[4], refs[5], refs[6],
            list(refs[7:7 + _NBUF]), list(refs[7 + _NBUF:7 + 2 * _NBUF]),
        ),
        out_type=jax.ShapeDtypeStruct((_B, _P, _D), jnp.float32),
        mesh=mesh,
        scratch_types=(
            [
                pltpu.VMEM((_RPB, _D), jnp.float32),
                pltpu.VMEM((_NBUF, 1, _RPS, _D), jnp.float32),
                pltpu.VMEM((1, _D), jnp.float32),
                pltpu.VMEM((2, 1, 1, _D), jnp.float32),
            ]
            + [pltpu.SemaphoreType.DMA] * (2 * _NBUF)
        ),
    )
    return f(patch, pos_table)


def kernel(patch, pos_table):
    return _sc_call(patch, pos_table)


# speedup vs baseline: 3.5175x; 2.2133x over previous
import jax
import jax.numpy as jnp
from jax import lax
from jax.experimental import pallas as pl
from jax.experimental.pallas import tpu as pltpu
from jax.experimental.pallas import tpu_sc as plsc

_NC, _NS, _L = 2, 16, 16
_NW = _NC * _NS
_B, _P, _D = 64, 577, 768
_RPW = (_P - 1) // _NW
_NGEN = _RPW // 2
_HB = _B // 2
_NCOL = _D // _L


def _in_copy(patch_t, ring_v, sems, p, h, s):
    return pltpu.make_async_copy(
        patch_t.at[pl.ds(p, 1), pl.ds(h * _HB, _HB)], ring_v.at[s], sems[s]
    )


def _out_copy(out_t, ring_v, sems, p, h, s):
    return pltpu.make_async_copy(
        ring_v.at[s], out_t.at[pl.ds(p, 1), pl.ds(h * _HB, _HB)], sems[s]
    )


def _add_slab(ring_v, tab_v, s, p_rel):
    @plsc.parallel_loop(0, _HB, 1, unroll=2)
    def j_body(j):
        for c0 in range(0, _NCOL, 8):
            vals = []
            for c in range(c0, c0 + 8):
                sl = pl.ds(c * _L, _L)
                tsl = pl.ds(p_rel * _D + c * _L, _L)
                vals.append(ring_v[s, 0, j, sl] + tab_v[tsl])
            for i, c in enumerate(range(c0, c0 + 8)):
                sl = pl.ds(c * _L, _L)
                ring_v[s, 0, j, sl] = vals[i]


def _sc_body(patch_t, table_f, out_t, tab_v, ring_v, ttab_v, tail_v,
             in_sems, out_sems):
    w = lax.axis_index("s") * _NC + lax.axis_index("c")
    p0 = w * _RPW

    pltpu.sync_copy(table_f.at[pl.ds(p0 * _D, _RPW * _D)], tab_v)

    for rp in range(2):
        for h in range(2):
            _in_copy(patch_t, ring_v, in_sems, p0 + rp, h, 2 * rp + h).start()

    def gen_body(g, carry):
        for rp in range(2):
            for h in range(2):
                s = 2 * rp + h
                _in_copy(patch_t, ring_v, in_sems, p0 + 2 * g + rp, h, s).wait()
                _add_slab(ring_v, tab_v, s, 2 * g + rp)
                _out_copy(out_t, ring_v, out_sems, p0 + 2 * g + rp, h, s).start()
        for rp in range(2):
            for h in range(2):
                s = 2 * rp + h
                _out_copy(out_t, ring_v, out_sems, p0 + 2 * g + rp, h, s).wait()
                _in_copy(
                    patch_t, ring_v, in_sems, p0 + 2 * (g + 1) + rp, h, s
                ).start()
        return carry

    lax.fori_loop(0, _NGEN - 1, gen_body, 0)

    gl = _NGEN - 1
    for rp in range(2):
        for h in range(2):
            s = 2 * rp + h
            _in_copy(patch_t, ring_v, in_sems, p0 + 2 * gl + rp, h, s).wait()
            _add_slab(ring_v, tab_v, s, 2 * gl + rp)
            _out_copy(out_t, ring_v, out_sems, p0 + 2 * gl + rp, h, s).start()
    for rp in range(2):
        for h in range(2):
            s = 2 * rp + h
            _out_copy(out_t, ring_v, out_sems, p0 + 2 * gl + rp, h, s).wait()

    @pl.when(w < 8)
    def _tail():
        pltpu.sync_copy(table_f.at[pl.ds((_P - 1) * _D, _D)], ttab_v)
        pltpu.sync_copy(
            patch_t.at[pl.ds(_P - 1, 1), pl.ds(w * 8, 8)], tail_v
        )
        for j in range(8):
            for c in range(_NCOL):
                sl = pl.ds(c * _L, _L)
                tail_v[0, j, sl] = tail_v[0, j, sl] + ttab_v[sl]
        pltpu.sync_copy(
            tail_v, out_t.at[pl.ds(_P - 1, 1), pl.ds(w * 8, 8)]
        )


@jax.jit
def _sc_call(patch, pos_table):
    patch_t = jnp.transpose(patch, (1, 0, 2))
    table_f = jnp.reshape(pos_table, (_P * _D,))
    mesh = plsc.VectorSubcoreMesh(core_axis_name="c", subcore_axis_name="s")
    f = pl.kernel(
        lambda *refs: _sc_body(
            refs[0], refs[1], refs[2], refs[3], refs[4], refs[5], refs[6],
            list(refs[7:11]), list(refs[11:15]),
        ),
        out_type=jax.ShapeDtypeStruct((_P, _B, _D), jnp.float32),
        mesh=mesh,
        scratch_types=(
            [
                pltpu.VMEM((_RPW * _D,), jnp.float32),
                pltpu.VMEM((4, 1, _HB, _D), jnp.float32),
                pltpu.VMEM((_D,), jnp.float32),
                pltpu.VMEM((1, 8, _D), jnp.float32),
            ]
            + [pltpu.SemaphoreType.DMA] * 8
        ),
    )
    out_t = f(patch_t, table_f)
    return jnp.transpose(out_t, (1, 0, 2))


def kernel(patch, pos_table):
    return _sc_call(patch, pos_table)


# speedup vs baseline: 4.4694x; 1.2706x over previous
import jax
import jax.numpy as jnp
from jax import lax
from jax.experimental import pallas as pl
from jax.experimental.pallas import tpu as pltpu
from jax.experimental.pallas import tpu_sc as plsc

_NC, _NS, _L = 2, 16, 16
_NW = _NC * _NS
_B, _P, _D = 64, 577, 768
_RPW = (_P - 1) // _NW
_NGEN = _RPW // 2
_HB = _B // 2
_NCOL = _D // _L


def _in_copy(patch_t, ring_v, sems, p, h, s):
    return pltpu.make_async_copy(
        patch_t.at[pl.ds(p, 1), pl.ds(h * _HB, _HB)], ring_v.at[s], sems[s]
    )


def _out_copy(out_t, ring_v, sems, p, h, s):
    return pltpu.make_async_copy(
        ring_v.at[s], out_t.at[pl.ds(p, 1), pl.ds(h * _HB, _HB)], sems[s]
    )


def _add_slab(ring_v, tab_v, s, p_rel):
    for c0 in range(0, _NCOL, 16):
        cols = range(c0, c0 + 16)
        tvals = [tab_v[pl.ds(p_rel * _D + c * _L, _L)] for c in cols]

        @plsc.parallel_loop(0, _HB, 1, unroll=2)
        def j_body(j):
            vals = []
            for i, c in enumerate(cols):
                sl = pl.ds(c * _L, _L)
                vals.append(ring_v[s, 0, j, sl] + tvals[i])
            for i, c in enumerate(cols):
                sl = pl.ds(c * _L, _L)
                ring_v[s, 0, j, sl] = vals[i]


def _sc_body(patch_t, table_f, out_t, tab_v, ring_v, ttab_v, tail_v,
             in_sems, out_sems):
    w = lax.axis_index("s") * _NC + lax.axis_index("c")
    p0 = w * _RPW

    pltpu.sync_copy(table_f.at[pl.ds(p0 * _D, _RPW * _D)], tab_v)

    for rp in range(2):
        for h in range(2):
            _in_copy(patch_t, ring_v, in_sems, p0 + rp, h, 2 * rp + h).start()

    def gen_body(g, carry):
        for rp in range(2):
            for h in range(2):
                s = 2 * rp + h
                _in_copy(patch_t, ring_v, in_sems, p0 + 2 * g + rp, h, s).wait()
                _add_slab(ring_v, tab_v, s, 2 * g + rp)
                _out_copy(out_t, ring_v, out_sems, p0 + 2 * g + rp, h, s).start()
        for rp in range(2):
            for h in range(2):
                s = 2 * rp + h
                _out_copy(out_t, ring_v, out_sems, p0 + 2 * g + rp, h, s).wait()
                _in_copy(
                    patch_t, ring_v, in_sems, p0 + 2 * (g + 1) + rp, h, s
                ).start()
        return carry

    lax.fori_loop(0, _NGEN - 1, gen_body, 0)

    gl = _NGEN - 1
    for rp in range(2):
        for h in range(2):
            s = 2 * rp + h
            _in_copy(patch_t, ring_v, in_sems, p0 + 2 * gl + rp, h, s).wait()
            _add_slab(ring_v, tab_v, s, 2 * gl + rp)
            _out_copy(out_t, ring_v, out_sems, p0 + 2 * gl + rp, h, s).start()
    for rp in range(2):
        for h in range(2):
            s = 2 * rp + h
            _out_copy(out_t, ring_v, out_sems, p0 + 2 * gl + rp, h, s).wait()

    @pl.when(w < 8)
    def _tail():
        pltpu.sync_copy(table_f.at[pl.ds((_P - 1) * _D, _D)], ttab_v)
        pltpu.sync_copy(
            patch_t.at[pl.ds(_P - 1, 1), pl.ds(w * 8, 8)], tail_v
        )
        for j in range(8):
            for c in range(_NCOL):
                sl = pl.ds(c * _L, _L)
                tail_v[0, j, sl] = tail_v[0, j, sl] + ttab_v[sl]
        pltpu.sync_copy(
            tail_v, out_t.at[pl.ds(_P - 1, 1), pl.ds(w * 8, 8)]
        )


@jax.jit
def _sc_call(patch, pos_table):
    patch_t = jnp.transpose(patch, (1, 0, 2))
    table_f = jnp.reshape(pos_table, (_P * _D,))
    mesh = plsc.VectorSubcoreMesh(core_axis_name="c", subcore_axis_name="s")
    f = pl.kernel(
        lambda *refs: _sc_body(
            refs[0], refs[1], refs[2], refs[3], refs[4], refs[5], refs[6],
            list(refs[7:11]), list(refs[11:15]),
        ),
        out_type=jax.ShapeDtypeStruct((_P, _B, _D), jnp.float32),
        mesh=mesh,
        scratch_types=(
            [
                pltpu.VMEM((_RPW * _D,), jnp.float32),
                pltpu.VMEM((4, 1, _HB, _D), jnp.float32),
                pltpu.VMEM((_D,), jnp.float32),
                pltpu.VMEM((1, 8, _D), jnp.float32),
            ]
            + [pltpu.SemaphoreType.DMA] * 8
        ),
    )
    out_t = f(patch_t, table_f)
    return jnp.transpose(out_t, (1, 0, 2))


def kernel(patch, pos_table):
    return _sc_call(patch, pos_table)


# speedup vs baseline: 4.5847x; 1.0258x over previous
import jax
import jax.numpy as jnp
from jax import lax
from jax.experimental import pallas as pl
from jax.experimental.pallas import tpu as pltpu
from jax.experimental.pallas import tpu_sc as plsc

_NC, _NS, _L = 2, 16, 16
_NW = _NC * _NS
_B, _P, _D = 64, 577, 768
_RPW = (_P - 1) // _NW
_NGEN = _RPW // 2
_HB = _B // 2
_NCOL = _D // _L


def _in_copy(patch_t, ring_v, sems, p, h, s):
    return pltpu.make_async_copy(
        patch_t.at[pl.ds(p, 1), pl.ds(h * _HB, _HB)], ring_v.at[s], sems[s]
    )


def _out_copy(out_t, ring_v, sems, p, h, s):
    return pltpu.make_async_copy(
        ring_v.at[s], out_t.at[pl.ds(p, 1), pl.ds(h * _HB, _HB)], sems[s]
    )


def _add_slab(ring_v, tab_v, s, p_rel):
    for c0 in range(0, _NCOL, 16):
        cols = range(c0, c0 + 16)
        tvals = [tab_v[pl.ds(p_rel * _D + c * _L, _L)] for c in cols]

        @plsc.parallel_loop(0, _HB, 1, unroll=2)
        def j_body(j):
            vals = []
            for i, c in enumerate(cols):
                sl = pl.ds(c * _L, _L)
                vals.append(ring_v[s, 0, j, sl] + tvals[i])
            for i, c in enumerate(cols):
                sl = pl.ds(c * _L, _L)
                ring_v[s, 0, j, sl] = vals[i]


def _sc_body(patch_t, table_f, out_t, tab_v, ring_v, ttab_v, tail_v,
             in_sems, out_sems):
    w = lax.axis_index("s") * _NC + lax.axis_index("c")
    p0 = w * _RPW

    pltpu.sync_copy(table_f.at[pl.ds(p0 * _D, _RPW * _D)], tab_v)

    for rp in range(2):
        for h in range(2):
            _in_copy(patch_t, ring_v, in_sems, p0 + rp, h, 2 * rp + h).start()

    def gen_body(g, carry):
        for rp in range(2):
            for h in range(2):
                s = 2 * rp + h
                _in_copy(patch_t, ring_v, in_sems, p0 + 2 * g + rp, h, s).wait()
                _add_slab(ring_v, tab_v, s, 2 * g + rp)
                _out_copy(out_t, ring_v, out_sems, p0 + 2 * g + rp, h, s).start()
            for h in range(2):
                s = 2 * rp + h
                _out_copy(out_t, ring_v, out_sems, p0 + 2 * g + rp, h, s).wait()
                _in_copy(
                    patch_t, ring_v, in_sems, p0 + 2 * (g + 1) + rp, h, s
                ).start()
        return carry

    lax.fori_loop(0, _NGEN - 1, gen_body, 0)

    gl = _NGEN - 1
    for rp in range(2):
        for h in range(2):
            s = 2 * rp + h
            _in_copy(patch_t, ring_v, in_sems, p0 + 2 * gl + rp, h, s).wait()
            _add_slab(ring_v, tab_v, s, 2 * gl + rp)
            _out_copy(out_t, ring_v, out_sems, p0 + 2 * gl + rp, h, s).start()
    for rp in range(2):
        for h in range(2):
            s = 2 * rp + h
            _out_copy(out_t, ring_v, out_sems, p0 + 2 * gl + rp, h, s).wait()

    @pl.when(w < 8)
    def _tail():
        pltpu.sync_copy(table_f.at[pl.ds((_P - 1) * _D, _D)], ttab_v)
        pltpu.sync_copy(
            patch_t.at[pl.ds(_P - 1, 1), pl.ds(w * 8, 8)], tail_v
        )
        for j in range(8):
            for c in range(_NCOL):
                sl = pl.ds(c * _L, _L)
                tail_v[0, j, sl] = tail_v[0, j, sl] + ttab_v[sl]
        pltpu.sync_copy(
            tail_v, out_t.at[pl.ds(_P - 1, 1), pl.ds(w * 8, 8)]
        )


@jax.jit
def _sc_call(patch, pos_table):
    patch_t = jnp.transpose(patch, (1, 0, 2))
    table_f = jnp.reshape(pos_table, (_P * _D,))
    mesh = plsc.VectorSubcoreMesh(core_axis_name="c", subcore_axis_name="s")
    f = pl.kernel(
        lambda *refs: _sc_body(
            refs[0], refs[1], refs[2], refs[3], refs[4], refs[5], refs[6],
            list(refs[7:11]), list(refs[11:15]),
        ),
        out_type=jax.ShapeDtypeStruct((_P, _B, _D), jnp.float32),
        mesh=mesh,
        scratch_types=(
            [
                pltpu.VMEM((_RPW * _D,), jnp.float32),
                pltpu.VMEM((4, 1, _HB, _D), jnp.float32),
                pltpu.VMEM((_D,), jnp.float32),
                pltpu.VMEM((1, 8, _D), jnp.float32),
            ]
            + [pltpu.SemaphoreType.DMA] * 8
        ),
    )
    out_t = f(patch_t, table_f)
    return jnp.transpose(out_t, (1, 0, 2))


def kernel(patch, pos_table):
    return _sc_call(patch, pos_table)
